# baseline (speedup 1.0000x reference)
import functools

import jax
import jax.numpy as jnp
from jax import lax
from jax.experimental import pallas as pl
from jax.experimental.pallas import tpu as pltpu
from jax.experimental.pallas import tpu_sc as plsc

_DIM = 64
_NC = 2
_NS = 16
_NW = _NC * _NS
_W = 8
_NBUF = 2


@functools.lru_cache(maxsize=None)
def _make_gather(n, vocab):
    assert n % (_NW * _W) == 0
    per_w = n // _NW
    n_win = per_w // _W
    assert n_win % _NBUF == 0
    mesh = plsc.VectorSubcoreMesh(core_axis_name="c", subcore_axis_name="s")

    @functools.partial(
        pl.kernel,
        out_type=jax.ShapeDtypeStruct((n, _DIM * _DIM), jnp.float32),
        mesh=mesh,
        scratch_types=[
            pltpu.VMEM((per_w,), jnp.int32),
            [pltpu.VMEM((_W, _DIM * _DIM), jnp.float32) for _ in range(_NBUF)],
            [pltpu.SemaphoreType.DMA for _ in range(_NBUF)],
            [pltpu.SemaphoreType.DMA for _ in range(_NBUF)],
        ],
    )
    def gather(maps_hbm, idx_hbm, out_hbm, idx_v, bufs, gsems, osems):
        wid = lax.axis_index("s") * _NC + lax.axis_index("c")
        base = wid * per_w
        pltpu.sync_copy(idx_hbm.at[pl.ds(base, per_w)], idx_v)

        def start_gather(w, buf_i):
            pltpu.async_copy(
                maps_hbm.at[idx_v.at[pl.ds(w * _W, _W)]], bufs[buf_i], gsems[buf_i]
            )

        def wait_gather(buf_i):
            pltpu.make_async_copy(
                maps_hbm.at[idx_v.at[pl.ds(0, _W)]], bufs[buf_i], gsems[buf_i]
            ).wait()

        def start_out(w, buf_i):
            pltpu.async_copy(
                bufs[buf_i],
                out_hbm.at[pl.ds(base + w * _W, _W)],
                osems[buf_i],
            )

        def wait_out(buf_i):
            pltpu.make_async_copy(
                bufs[buf_i], out_hbm.at[pl.ds(base, _W)], osems[buf_i]
            ).wait()

        start_gather(0, 0)

        def body(g, carry):
            for bi in range(_NBUF):
                w = g * _NBUF + bi
                wait_gather(bi)
                start_out(w, bi)
                nxt = (bi + 1) % _NBUF
                if bi + 1 < _NBUF:
                    @pl.when(g >= 1)
                    def _():
                        wait_out(nxt)
                else:
                    wait_out(nxt)
                @pl.when(w + 1 < n_win)
                def _():
                    start_gather(w + 1, nxt)
            return carry

        lax.fori_loop(0, n_win // _NBUF, body, 0)
        wait_out((n_win - 1) % _NBUF)

    return gather


def kernel(position_ids, maps):
    b, s = position_ids.shape
    vocab = maps.shape[0]
    idx = position_ids.reshape(b * s)
    maps2 = maps.reshape(vocab, _DIM * _DIM)
    out = _make_gather(b * s, vocab)(maps2, idx)
    return out.reshape(b, s, _DIM, _DIM)
